# baseline (speedup 1.0000x reference)
import jax
import jax.numpy as jnp
import numpy as np
from jax.experimental import pallas as pl


def _moe_kernel(h2_ref, y_ref, w_ref, eu1c_ref, ev1_ref, eb1_ref,
                eu2_ref, ev2s_ref, eb2_ref, out_ref, *, E, R, FF):
    h2 = h2_ref[...]
    w = w_ref[...]
    z1all = jnp.dot(h2, eu1c_ref[...], preferred_element_type=jnp.float32)
    parts = []
    for e in range(E):
        z = jnp.dot(z1all[:, e * R:(e + 1) * R], ev1_ref[e],
                    preferred_element_type=jnp.float32) + eb1_ref[e]
        z = jnp.maximum(z, 0.0)
        z2 = jnp.dot(z, eu2_ref[e], preferred_element_type=jnp.float32)
        parts.append(z2 * w[:, e:e + 1])
    z2w = jnp.concatenate(parts, axis=1)
    out_ref[...] = (y_ref[...]
                    + jnp.dot(z2w, ev2s_ref[...],
                              preferred_element_type=jnp.float32)
                    + jnp.dot(w, eb2_ref[...],
                              preferred_element_type=jnp.float32))


def kernel(x, u_qkv, v_qkv, b_qkv, u_attn, v_attn, u_out, v_out, b_out,
           n1g, n1b, n2g, n2b, gate_w, gate_b,
           eu1, ev1, eb1, eu2, ev2, eb2):
    B, S, D = x.shape
    H, HD, R = u_attn.shape
    E = gate_w.shape[1]
    FF = ev1.shape[2]

    def _layernorm(t, g, b):
        m = t.mean(-1, keepdims=True)
        v = ((t - m) ** 2).mean(-1, keepdims=True)
        return (t - m) / jnp.sqrt(v + 1e-5) * g + b

    h = _layernorm(x, n1g, n1b)
    qkv = h @ u_qkv @ v_qkv + b_qkv
    q, k, v = jnp.split(qkv, 3, axis=-1)

    def rs(t):
        return t.reshape(B, S, H, HD).transpose(0, 2, 1, 3)

    q, k, v = rs(q), rs(k), rs(v)
    q_low = jnp.einsum('bhsd,hdr->bhsr', q, u_attn)
    k_low = jnp.einsum('bhsd,hdr->bhsr', k, u_attn)
    scores = jnp.einsum('bhsr,bhtr->bhst', q_low, k_low) * (1.0 / np.sqrt(R))
    attn = jax.nn.softmax(scores, axis=-1)
    v_low = jnp.einsum('bhsd,hdr->bhsr', v, u_attn)
    ctx_low = jnp.einsum('bhst,bhtr->bhsr', attn, v_low)
    ctx = jnp.einsum('bhsr,hrd->bhsd', ctx_low, v_attn)
    ctx = ctx.transpose(0, 2, 1, 3).reshape(B, S, D)
    attn_out = ctx @ u_out @ v_out + b_out
    y = x + attn_out
    h2 = _layernorm(y, n2g, n2b)
    logits = h2 @ gate_w + gate_b
    probs = jax.nn.softmax(logits, axis=-1)
    tkp, tki = jax.lax.top_k(probs, 2)
    tkp = tkp / tkp.sum(-1, keepdims=True)
    w = jnp.sum((tki[..., None] == jnp.arange(E)[None, None, None, :]
                 ).astype(jnp.float32) * tkp[..., None], axis=2)

    eu1c = jnp.transpose(eu1, (1, 0, 2)).reshape(D, E * R)
    ev2s = ev2.reshape(E * R, D)

    SB = 1024
    import functools
    out = pl.pallas_call(
        functools.partial(_moe_kernel, E=E, R=R, FF=FF),
        grid=(S // SB,),
        in_specs=[
            pl.BlockSpec((SB, D), lambda i: (i, 0)),
            pl.BlockSpec((SB, D), lambda i: (i, 0)),
            pl.BlockSpec((SB, E), lambda i: (i, 0)),
            pl.BlockSpec((D, E * R), lambda i: (0, 0)),
            pl.BlockSpec((E, R, FF), lambda i: (0, 0, 0)),
            pl.BlockSpec((E, 1, FF), lambda i: (0, 0, 0)),
            pl.BlockSpec((E, FF, R), lambda i: (0, 0, 0)),
            pl.BlockSpec((E * R, D), lambda i: (0, 0)),
            pl.BlockSpec((E, D), lambda i: (0, 0)),
        ],
        out_specs=pl.BlockSpec((SB, D), lambda i: (i, 0)),
        out_shape=jax.ShapeDtypeStruct((S, D), jnp.float32),
    )(h2.reshape(S, D), y.reshape(S, D), w.reshape(S, E),
      eu1c, ev1, eb1.reshape(E, 1, FF), eu2, ev2s, eb2)

    return out.reshape(B, S, D)


# speedup vs baseline: 1.0296x; 1.0296x over previous
import jax
import jax.numpy as jnp
import numpy as np
from jax.experimental import pallas as pl


def _moe_kernel(h2_ref, y_ref, w_ref, eu1c_ref, ev1_ref, eb1_ref,
                eu2_ref, ev2s_ref, eb2_ref, out_ref, *, E, R, FF):
    h2 = h2_ref[...]
    w = w_ref[...]
    z1all = jnp.dot(h2, eu1c_ref[...], preferred_element_type=jnp.float32)
    parts = []
    for e in range(E):
        z = jnp.dot(z1all[:, e * R:(e + 1) * R], ev1_ref[e],
                    preferred_element_type=jnp.float32) + eb1_ref[e]
        z = jnp.maximum(z, 0.0)
        z2 = jnp.dot(z, eu2_ref[e], preferred_element_type=jnp.float32)
        parts.append(z2 * w[:, e:e + 1])
    z2w = jnp.concatenate(parts, axis=1)
    out_ref[...] = (y_ref[...]
                    + jnp.dot(z2w, ev2s_ref[...],
                              preferred_element_type=jnp.float32)
                    + jnp.dot(w, eb2_ref[...],
                              preferred_element_type=jnp.float32))


def kernel(x, u_qkv, v_qkv, b_qkv, u_attn, v_attn, u_out, v_out, b_out,
           n1g, n1b, n2g, n2b, gate_w, gate_b,
           eu1, ev1, eb1, eu2, ev2, eb2):
    B, S, D = x.shape
    H, HD, R = u_attn.shape
    E = gate_w.shape[1]
    FF = ev1.shape[2]

    def _layernorm(t, g, b):
        m = t.mean(-1, keepdims=True)
        v = ((t - m) ** 2).mean(-1, keepdims=True)
        return (t - m) / jnp.sqrt(v + 1e-5) * g + b

    h = _layernorm(x, n1g, n1b)
    qkv = h @ u_qkv @ v_qkv + b_qkv
    q, k, v = jnp.split(qkv, 3, axis=-1)

    def rs(t):
        return t.reshape(B, S, H, HD).transpose(0, 2, 1, 3)

    q, k, v = rs(q), rs(k), rs(v)
    q_low = jnp.einsum('bhsd,hdr->bhsr', q, u_attn)
    k_low = jnp.einsum('bhsd,hdr->bhsr', k, u_attn)
    scores = jnp.einsum('bhsr,bhtr->bhst', q_low, k_low) * (1.0 / np.sqrt(R))
    attn = jax.nn.softmax(scores, axis=-1)
    v_low = jnp.einsum('bhsd,hdr->bhsr', v, u_attn)
    ctx_low = jnp.einsum('bhst,bhtr->bhsr', attn, v_low)
    ctx = jnp.einsum('bhsr,hrd->bhsd', ctx_low, v_attn)
    ctx = ctx.transpose(0, 2, 1, 3).reshape(B, S, D)
    attn_out = ctx @ u_out @ v_out + b_out
    y = x + attn_out
    h2 = _layernorm(y, n2g, n2b)
    logits = h2 @ gate_w + gate_b
    probs = jax.nn.softmax(logits, axis=-1)
    tkp, tki = jax.lax.top_k(probs, 2)
    tkp = tkp / tkp.sum(-1, keepdims=True)
    w = jnp.sum((tki[..., None] == jnp.arange(E)[None, None, None, :]
                 ).astype(jnp.float32) * tkp[..., None], axis=2)

    eu1c = jnp.transpose(eu1, (1, 0, 2)).reshape(D, E * R)
    ev2s = ev2.reshape(E * R, D)

    SB = 256
    import functools
    out = pl.pallas_call(
        functools.partial(_moe_kernel, E=E, R=R, FF=FF),
        grid=(S // SB,),
        in_specs=[
            pl.BlockSpec((SB, D), lambda i: (i, 0)),
            pl.BlockSpec((SB, D), lambda i: (i, 0)),
            pl.BlockSpec((SB, E), lambda i: (i, 0)),
            pl.BlockSpec((D, E * R), lambda i: (0, 0)),
            pl.BlockSpec((E, R, FF), lambda i: (0, 0, 0)),
            pl.BlockSpec((E, 1, FF), lambda i: (0, 0, 0)),
            pl.BlockSpec((E, FF, R), lambda i: (0, 0, 0)),
            pl.BlockSpec((E * R, D), lambda i: (0, 0)),
            pl.BlockSpec((E, D), lambda i: (0, 0)),
        ],
        out_specs=pl.BlockSpec((SB, D), lambda i: (i, 0)),
        out_shape=jax.ShapeDtypeStruct((S, D), jnp.float32),
    )(h2.reshape(S, D), y.reshape(S, D), w.reshape(S, E),
      eu1c, ev1, eb1.reshape(E, 1, FF), eu2, ev2s, eb2)

    return out.reshape(B, S, D)


# speedup vs baseline: 1.0324x; 1.0028x over previous
import jax
import jax.numpy as jnp
import numpy as np
from jax.experimental import pallas as pl


def _moe_kernel(h2_ref, y_ref, w_ref, eu1c_ref, ev1_ref, eb1_ref,
                eu2_ref, ev2s_ref, eb2_ref, out_ref, *, E, R, FF):
    h2 = h2_ref[...]
    w = w_ref[...]
    z1all = jnp.dot(h2, eu1c_ref[...], preferred_element_type=jnp.float32)
    z1b = z1all.astype(jnp.bfloat16)
    parts = []
    for e in range(E):
        z = jnp.dot(z1b[:, e * R:(e + 1) * R],
                    ev1_ref[e].astype(jnp.bfloat16),
                    preferred_element_type=jnp.float32) + eb1_ref[e]
        z = jnp.maximum(z, 0.0).astype(jnp.bfloat16)
        z2 = jnp.dot(z, eu2_ref[e].astype(jnp.bfloat16),
                     preferred_element_type=jnp.float32)
        parts.append(z2 * w[:, e:e + 1])
    z2w = jnp.concatenate(parts, axis=1)
    out_ref[...] = (y_ref[...]
                    + jnp.dot(z2w, ev2s_ref[...],
                              preferred_element_type=jnp.float32)
                    + jnp.dot(w, eb2_ref[...],
                              preferred_element_type=jnp.float32))


def kernel(x, u_qkv, v_qkv, b_qkv, u_attn, v_attn, u_out, v_out, b_out,
           n1g, n1b, n2g, n2b, gate_w, gate_b,
           eu1, ev1, eb1, eu2, ev2, eb2):
    B, S, D = x.shape
    H, HD, R = u_attn.shape
    E = gate_w.shape[1]
    FF = ev1.shape[2]

    def _layernorm(t, g, b):
        m = t.mean(-1, keepdims=True)
        v = ((t - m) ** 2).mean(-1, keepdims=True)
        return (t - m) / jnp.sqrt(v + 1e-5) * g + b

    h = _layernorm(x, n1g, n1b)
    qkv = h @ u_qkv @ v_qkv + b_qkv
    q, k, v = jnp.split(qkv, 3, axis=-1)

    def rs(t):
        return t.reshape(B, S, H, HD).transpose(0, 2, 1, 3)

    q, k, v = rs(q), rs(k), rs(v)
    q_low = jnp.einsum('bhsd,hdr->bhsr', q, u_attn)
    k_low = jnp.einsum('bhsd,hdr->bhsr', k, u_attn)
    scores = jnp.einsum('bhsr,bhtr->bhst', q_low, k_low) * (1.0 / np.sqrt(R))
    attn = jax.nn.softmax(scores, axis=-1)
    v_low = jnp.einsum('bhsd,hdr->bhsr', v, u_attn)
    ctx_low = jnp.einsum('bhst,bhtr->bhsr', attn, v_low)
    ctx = jnp.einsum('bhsr,hrd->bhsd', ctx_low, v_attn)
    ctx = ctx.transpose(0, 2, 1, 3).reshape(B, S, D)
    attn_out = ctx @ u_out @ v_out + b_out
    y = x + attn_out
    h2 = _layernorm(y, n2g, n2b)
    logits = h2 @ gate_w + gate_b
    probs = jax.nn.softmax(logits, axis=-1)
    tkp, tki = jax.lax.top_k(probs, 2)
    tkp = tkp / tkp.sum(-1, keepdims=True)
    w = jnp.sum((tki[..., None] == jnp.arange(E)[None, None, None, :]
                 ).astype(jnp.float32) * tkp[..., None], axis=2)

    eu1c = jnp.transpose(eu1, (1, 0, 2)).reshape(D, E * R)
    ev2s = ev2.reshape(E * R, D)

    SB = 512
    import functools
    out = pl.pallas_call(
        functools.partial(_moe_kernel, E=E, R=R, FF=FF),
        grid=(S // SB,),
        in_specs=[
            pl.BlockSpec((SB, D), lambda i: (i, 0)),
            pl.BlockSpec((SB, D), lambda i: (i, 0)),
            pl.BlockSpec((SB, E), lambda i: (i, 0)),
            pl.BlockSpec((D, E * R), lambda i: (0, 0)),
            pl.BlockSpec((E, R, FF), lambda i: (0, 0, 0)),
            pl.BlockSpec((E, 1, FF), lambda i: (0, 0, 0)),
            pl.BlockSpec((E, FF, R), lambda i: (0, 0, 0)),
            pl.BlockSpec((E * R, D), lambda i: (0, 0)),
            pl.BlockSpec((E, D), lambda i: (0, 0)),
        ],
        out_specs=pl.BlockSpec((SB, D), lambda i: (i, 0)),
        out_shape=jax.ShapeDtypeStruct((S, D), jnp.float32),
    )(h2.reshape(S, D), y.reshape(S, D), w.reshape(S, E),
      eu1c, ev1, eb1.reshape(E, 1, FF), eu2, ev2s, eb2)

    return out.reshape(B, S, D)


# speedup vs baseline: 1.0410x; 1.0083x over previous
import jax
import jax.numpy as jnp
import numpy as np
from jax.experimental import pallas as pl


def _moe_kernel(y_ref, w_ref, n2g_ref, n2b_ref, eu1c_ref, ev1_ref, eb1_ref,
                eu2_ref, ev2s_ref, eb2_ref, out_ref, *, E, R, FF):
    y = y_ref[...]
    m = jnp.mean(y, axis=-1, keepdims=True)
    var = jnp.mean((y - m) ** 2, axis=-1, keepdims=True)
    h2 = (y - m) / jnp.sqrt(var + 1e-5) * n2g_ref[...] + n2b_ref[...]
    w = w_ref[...]
    z1all = jnp.dot(h2, eu1c_ref[...], preferred_element_type=jnp.float32)
    z1b = z1all.astype(jnp.bfloat16)
    parts = []
    for e in range(E):
        z = jnp.dot(z1b[:, e * R:(e + 1) * R],
                    ev1_ref[e].astype(jnp.bfloat16),
                    preferred_element_type=jnp.float32) + eb1_ref[e]
        z = jnp.maximum(z, 0.0).astype(jnp.bfloat16)
        z2 = jnp.dot(z, eu2_ref[e].astype(jnp.bfloat16),
                     preferred_element_type=jnp.float32)
        parts.append(z2 * w[:, e:e + 1])
    z2w = jnp.concatenate(parts, axis=1)
    out_ref[...] = (y
                    + jnp.dot(z2w, ev2s_ref[...],
                              preferred_element_type=jnp.float32)
                    + jnp.dot(w, eb2_ref[...],
                              preferred_element_type=jnp.float32))


def kernel(x, u_qkv, v_qkv, b_qkv, u_attn, v_attn, u_out, v_out, b_out,
           n1g, n1b, n2g, n2b, gate_w, gate_b,
           eu1, ev1, eb1, eu2, ev2, eb2):
    B, S, D = x.shape
    H, HD, R = u_attn.shape
    E = gate_w.shape[1]
    FF = ev1.shape[2]

    def _layernorm(t, g, b):
        m = t.mean(-1, keepdims=True)
        v = ((t - m) ** 2).mean(-1, keepdims=True)
        return (t - m) / jnp.sqrt(v + 1e-5) * g + b

    h = _layernorm(x, n1g, n1b)
    qkv = h @ u_qkv @ v_qkv + b_qkv
    q, k, v = jnp.split(qkv, 3, axis=-1)

    def rs(t):
        return t.reshape(B, S, H, HD).transpose(0, 2, 1, 3)

    q, k, v = rs(q), rs(k), rs(v)
    q_low = jnp.einsum('bhsd,hdr->bhsr', q, u_attn)
    k_low = jnp.einsum('bhsd,hdr->bhsr', k, u_attn)
    scores = jnp.einsum('bhsr,bhtr->bhst', q_low, k_low) * (1.0 / np.sqrt(R))
    attn = jax.nn.softmax(scores, axis=-1)
    v_low = jnp.einsum('bhsd,hdr->bhsr', v, u_attn)
    ctx_low = jnp.einsum('bhst,bhtr->bhsr', attn, v_low)
    ctx = jnp.einsum('bhsr,hrd->bhsd', ctx_low, v_attn)
    ctx = ctx.transpose(0, 2, 1, 3).reshape(B, S, D)
    attn_out = ctx @ u_out @ v_out + b_out
    y = x + attn_out
    h2 = _layernorm(y, n2g, n2b)
    logits = h2 @ gate_w + gate_b
    probs = jax.nn.softmax(logits, axis=-1)
    tkp, tki = jax.lax.top_k(probs, 2)
    tkp = tkp / tkp.sum(-1, keepdims=True)
    w = jnp.sum((tki[..., None] == jnp.arange(E)[None, None, None, :]
                 ).astype(jnp.float32) * tkp[..., None], axis=2)

    eu1c = jnp.transpose(eu1, (1, 0, 2)).reshape(D, E * R)
    ev2s = ev2.reshape(E * R, D)

    SB = 512
    import functools
    out = pl.pallas_call(
        functools.partial(_moe_kernel, E=E, R=R, FF=FF),
        grid=(S // SB,),
        in_specs=[
            pl.BlockSpec((SB, D), lambda i: (i, 0)),
            pl.BlockSpec((SB, E), lambda i: (i, 0)),
            pl.BlockSpec((1, D), lambda i: (0, 0)),
            pl.BlockSpec((1, D), lambda i: (0, 0)),
            pl.BlockSpec((D, E * R), lambda i: (0, 0)),
            pl.BlockSpec((E, R, FF), lambda i: (0, 0, 0)),
            pl.BlockSpec((E, 1, FF), lambda i: (0, 0, 0)),
            pl.BlockSpec((E, FF, R), lambda i: (0, 0, 0)),
            pl.BlockSpec((E * R, D), lambda i: (0, 0)),
            pl.BlockSpec((E, D), lambda i: (0, 0)),
        ],
        out_specs=pl.BlockSpec((SB, D), lambda i: (i, 0)),
        out_shape=jax.ShapeDtypeStruct((S, D), jnp.float32),
    )(y.reshape(S, D), w.reshape(S, E), n2g.reshape(1, D),
      n2b.reshape(1, D), eu1c, ev1, eb1.reshape(E, 1, FF), eu2, ev2s, eb2)

    return out.reshape(B, S, D)


# speedup vs baseline: 1.0424x; 1.0013x over previous
import jax
import jax.numpy as jnp
import numpy as np
from jax.experimental import pallas as pl


def _moe_kernel(y_ref, w_ref, n2g_ref, n2b_ref, eu1c_ref, ev1_ref, eb1_ref,
                eu2_ref, ev2s_ref, eb2_ref, out_ref, *, E, R, FF):
    y = y_ref[...]
    m = jnp.mean(y, axis=-1, keepdims=True)
    var = jnp.mean((y - m) ** 2, axis=-1, keepdims=True)
    h2 = (y - m) / jnp.sqrt(var + 1e-5) * n2g_ref[...] + n2b_ref[...]
    w = w_ref[...]
    z1all = jnp.dot(h2, eu1c_ref[...], preferred_element_type=jnp.float32)
    parts = []
    for e in range(E):
        z = jnp.dot(z1all[:, e * R:(e + 1) * R], ev1_ref[e],
                    preferred_element_type=jnp.float32) + eb1_ref[e]
        z = jnp.maximum(z, 0.0)
        z2 = jnp.dot(z, eu2_ref[e], preferred_element_type=jnp.float32)
        parts.append(z2 * w[:, e:e + 1])
    z2w = jnp.concatenate(parts, axis=1)
    out_ref[...] = (y
                    + jnp.dot(z2w, ev2s_ref[...],
                              preferred_element_type=jnp.float32)
                    + jnp.dot(w, eb2_ref[...],
                              preferred_element_type=jnp.float32))


def kernel(x, u_qkv, v_qkv, b_qkv, u_attn, v_attn, u_out, v_out, b_out,
           n1g, n1b, n2g, n2b, gate_w, gate_b,
           eu1, ev1, eb1, eu2, ev2, eb2):
    B, S, D = x.shape
    H, HD, R = u_attn.shape
    E = gate_w.shape[1]
    FF = ev1.shape[2]

    def _layernorm(t, g, b):
        m = t.mean(-1, keepdims=True)
        v = ((t - m) ** 2).mean(-1, keepdims=True)
        return (t - m) / jnp.sqrt(v + 1e-5) * g + b

    h = _layernorm(x, n1g, n1b)
    qkv = h @ u_qkv @ v_qkv + b_qkv
    q, k, v = jnp.split(qkv, 3, axis=-1)

    def rs(t):
        return t.reshape(B, S, H, HD).transpose(0, 2, 1, 3)

    q, k, v = rs(q), rs(k), rs(v)
    q_low = jnp.einsum('bhsd,hdr->bhsr', q, u_attn)
    k_low = jnp.einsum('bhsd,hdr->bhsr', k, u_attn)
    scores = jnp.einsum('bhsr,bhtr->bhst', q_low, k_low) * (1.0 / np.sqrt(R))
    attn = jax.nn.softmax(scores, axis=-1)
    v_low = jnp.einsum('bhsd,hdr->bhsr', v, u_attn)
    ctx_low = jnp.einsum('bhst,bhtr->bhsr', attn, v_low)
    ctx = jnp.einsum('bhsr,hrd->bhsd', ctx_low, v_attn)
    ctx = ctx.transpose(0, 2, 1, 3).reshape(B, S, D)
    attn_out = ctx @ u_out @ v_out + b_out
    y = x + attn_out
    h2 = _layernorm(y, n2g, n2b)
    logits = h2 @ gate_w + gate_b
    probs = jax.nn.softmax(logits, axis=-1)
    tkp, tki = jax.lax.top_k(probs, 2)
    tkp = tkp / tkp.sum(-1, keepdims=True)
    w = jnp.sum((tki[..., None] == jnp.arange(E)[None, None, None, :]
                 ).astype(jnp.float32) * tkp[..., None], axis=2)

    eu1c = jnp.transpose(eu1, (1, 0, 2)).reshape(D, E * R)
    ev2s = ev2.reshape(E * R, D)

    SB = 512
    import functools
    out = pl.pallas_call(
        functools.partial(_moe_kernel, E=E, R=R, FF=FF),
        grid=(S // SB,),
        in_specs=[
            pl.BlockSpec((SB, D), lambda i: (i, 0)),
            pl.BlockSpec((SB, E), lambda i: (i, 0)),
            pl.BlockSpec((1, D), lambda i: (0, 0)),
            pl.BlockSpec((1, D), lambda i: (0, 0)),
            pl.BlockSpec((D, E * R), lambda i: (0, 0)),
            pl.BlockSpec((E, R, FF), lambda i: (0, 0, 0)),
            pl.BlockSpec((E, 1, FF), lambda i: (0, 0, 0)),
            pl.BlockSpec((E, FF, R), lambda i: (0, 0, 0)),
            pl.BlockSpec((E * R, D), lambda i: (0, 0)),
            pl.BlockSpec((E, D), lambda i: (0, 0)),
        ],
        out_specs=pl.BlockSpec((SB, D), lambda i: (i, 0)),
        out_shape=jax.ShapeDtypeStruct((S, D), jnp.float32),
    )(y.reshape(S, D), w.reshape(S, E), n2g.reshape(1, D),
      n2b.reshape(1, D), eu1c, ev1, eb1.reshape(E, 1, FF), eu2, ev2s, eb2)

    return out.reshape(B, S, D)
